# baseline (speedup 1.0000x reference)
import functools
import jax
import jax.numpy as jnp
from jax import lax
from jax.experimental import pallas as pl
from jax.experimental.pallas import tpu as pltpu
from jax.experimental.pallas import tpu_sc as plsc

NUM_CLASSES_ = 1000
N_ = 16384
NW_ = 32
RPW_ = N_ // NW_
CH_ = 64
NCHUNK_ = RPW_ // CH_


def _sc_body(x_hbm, out_hbm, idx_v, rows_v, sem):
    c = lax.axis_index("c")
    s = lax.axis_index("s")
    wid = s * 2 + c
    base = wid * RPW_

    pltpu.sync_copy(x_hbm.at[pl.ds(base, RPW_)], idx_v)

    zeros16 = jnp.zeros((16,), jnp.float32)
    ones16 = jnp.ones((16,), jnp.float32)
    lane = lax.iota(jnp.int32, 16)
    tail_mask = lane < (NUM_CLASSES_ - 992)

    def _zrow(r, _):
        for l in range(62):
            rows_v[r, pl.ds(l * 16, 16)] = zeros16
        rsplat = jnp.full((16,), 0, jnp.int32) + r
        plsc.store_scatter(rows_v, [rsplat, 992 + lane], zeros16,
                           mask=tail_mask)
        return _

    lax.fori_loop(0, CH_, _zrow, 0)

    def _chunk(k, _):
        def _scat(g, val):
            rowi = g * 16 + lane
            coli = idx_v[pl.ds(k * CH_ + g * 16, 16)]
            plsc.store_scatter(rows_v, [rowi, coli], val)

        def _set(g, __):
            _scat(g, ones16)
            return __

        lax.fori_loop(0, CH_ // 16, _set, 0)
        pltpu.sync_copy(rows_v, out_hbm.at[pl.ds(base + k * CH_, CH_)])

        def _clr(g, __):
            _scat(g, zeros16)
            return __

        lax.fori_loop(0, CH_ // 16, _clr, 0)
        return _

    lax.fori_loop(0, NCHUNK_, _chunk, 0)


def kernel(x):
    mesh = plsc.VectorSubcoreMesh(core_axis_name="c", subcore_axis_name="s")
    k = functools.partial(
        pl.kernel,
        mesh=mesh,
        out_type=jax.ShapeDtypeStruct((N_, NUM_CLASSES_), jnp.float32),
        scratch_types=[
            pltpu.VMEM((RPW_,), jnp.int32),
            pltpu.VMEM((CH_, NUM_CLASSES_), jnp.float32),
            pltpu.SemaphoreType.DMA,
        ],
        compiler_params=pltpu.CompilerParams(needs_layout_passes=False),
    )(_sc_body)
    return k(x.astype(jnp.int32))
